# Initial kernel scaffold; baseline (speedup 1.0000x reference)
import jax
import jax.numpy as jnp
from jax.experimental import pallas as pl


def kernel(x):
    raise NotImplementedError("write your pallas kernel here")


# speedup vs baseline: 1.4909x; 1.4909x over previous
import jax
import jax.numpy as jnp
from jax.experimental import pallas as pl
from jax.experimental.pallas import tpu as pltpu

S_BLK = 256
D_BLK = 512


def _scan_body(x_ref, o_ref, carry_ref):
    s = pl.program_id(2)

    @pl.when(s == 0)
    def _():
        carry_ref[...] = jnp.zeros_like(carry_ref)

    acc = x_ref[0]
    k = 1
    while k < S_BLK:
        shifted = jnp.concatenate(
            [jnp.zeros((k, D_BLK), jnp.float32), acc[:-k, :]], axis=0)
        acc = acc + shifted
        k *= 2
    out = acc + carry_ref[...]
    o_ref[0] = out
    carry_ref[...] = out[S_BLK - 1:S_BLK, :]


def kernel(x):
    B, S, D = x.shape
    grid = (B, D // D_BLK, S // S_BLK)
    return pl.pallas_call(
        _scan_body,
        grid=grid,
        in_specs=[pl.BlockSpec((1, S_BLK, D_BLK), lambda b, d, s: (b, s, d))],
        out_specs=pl.BlockSpec((1, S_BLK, D_BLK), lambda b, d, s: (b, s, d)),
        out_shape=jax.ShapeDtypeStruct(x.shape, x.dtype),
        scratch_shapes=[pltpu.VMEM((1, D_BLK), jnp.float32)],
        compiler_params=pltpu.CompilerParams(
            dimension_semantics=("parallel", "parallel", "arbitrary")),
    )(x)


# speedup vs baseline: 1.5316x; 1.0274x over previous
import jax
import jax.numpy as jnp
from jax.experimental import pallas as pl
from jax.experimental.pallas import tpu as pltpu

S_BLK = 256
D_BLK = 512


def _scan_body(x_ref, o_ref, carry_ref):
    s = pl.program_id(2)

    @pl.when(s == 0)
    def _():
        carry_ref[...] = jnp.zeros_like(carry_ref)

    xb = x_ref[0]
    ri = jax.lax.broadcasted_iota(jnp.int32, (S_BLK, S_BLK), 0)
    ci = jax.lax.broadcasted_iota(jnp.int32, (S_BLK, S_BLK), 1)
    tri = (ri >= ci).astype(jnp.float32)
    local = jnp.dot(tri, xb, preferred_element_type=jnp.float32)
    out = local + carry_ref[...]
    o_ref[0] = out
    carry_ref[...] = out[S_BLK - 1:S_BLK, :]


def kernel(x):
    B, S, D = x.shape
    grid = (B, D // D_BLK, S // S_BLK)
    return pl.pallas_call(
        _scan_body,
        grid=grid,
        in_specs=[pl.BlockSpec((1, S_BLK, D_BLK), lambda b, d, s: (b, s, d))],
        out_specs=pl.BlockSpec((1, S_BLK, D_BLK), lambda b, d, s: (b, s, d)),
        out_shape=jax.ShapeDtypeStruct(x.shape, x.dtype),
        scratch_shapes=[pltpu.VMEM((1, D_BLK), jnp.float32)],
        compiler_params=pltpu.CompilerParams(
            dimension_semantics=("parallel", "parallel", "arbitrary")),
    )(x)


# speedup vs baseline: 3.2092x; 2.0953x over previous
import jax
import jax.numpy as jnp
from jax.experimental import pallas as pl
from jax.experimental.pallas import tpu as pltpu

S_BLK = 256
D_BLK = 2048


def _scan_body(x_ref, o_ref, carry_ref):
    s = pl.program_id(2)

    @pl.when(s == 0)
    def _():
        carry_ref[...] = jnp.zeros_like(carry_ref)

    xb = x_ref[0]
    ri = jax.lax.broadcasted_iota(jnp.int32, (S_BLK, S_BLK), 0)
    ci = jax.lax.broadcasted_iota(jnp.int32, (S_BLK, S_BLK), 1)
    tri = (ri >= ci).astype(jnp.float32)
    local = jnp.dot(tri, xb, preferred_element_type=jnp.float32)
    out = local + carry_ref[...]
    o_ref[0] = out
    carry_ref[...] = out[S_BLK - 1:S_BLK, :]


def kernel(x):
    B, S, D = x.shape
    grid = (B, D // D_BLK, S // S_BLK)
    return pl.pallas_call(
        _scan_body,
        grid=grid,
        in_specs=[pl.BlockSpec((1, S_BLK, D_BLK), lambda b, d, s: (b, s, d))],
        out_specs=pl.BlockSpec((1, S_BLK, D_BLK), lambda b, d, s: (b, s, d)),
        out_shape=jax.ShapeDtypeStruct(x.shape, x.dtype),
        scratch_shapes=[pltpu.VMEM((1, D_BLK), jnp.float32)],
        compiler_params=pltpu.CompilerParams(
            dimension_semantics=("parallel", "parallel", "arbitrary")),
    )(x)
